# baseline (device time: 537010 ns/iter reference)
import jax
import jax.numpy as jnp
from jax import lax
from jax.experimental import pallas as pl
from jax.experimental.pallas import tpu as pltpu

T = 1024
D = 2048
HALF = 16384
CH = 512
NC = HALF // CH
LMAX = 8.0


def kernel(x, W):
    xb = x.astype(jnp.bfloat16)

    def body(xb_ref, wblk, recv_hbm, ebf_hbm, stot_ref,
             stats_recv, ebf_all, s_ref, s_out_vmem,
             send_sems, recv_sems, st_send_sem, st_recv_sem,
             estore_sems, sstore_sem):
        i = pl.program_id(0)
        xi = lax.axis_index("x")
        yi = lax.axis_index("y")
        zi = lax.axis_index("z")
        partner = (xi, yi, 1 - zi)

        @pl.when(i == 0)
        def _():
            barrier = pltpu.get_barrier_semaphore()
            pl.semaphore_signal(barrier, inc=1, device_id=partner,
                                device_id_type=pl.DeviceIdType.MESH)
            pl.semaphore_wait(barrier, 1)
            s_ref[...] = jnp.zeros((T, 1), jnp.float32)

        wb = wblk[...].astype(jnp.bfloat16)
        l = jnp.dot(xb_ref[...], wb, preferred_element_type=jnp.float32)
        e = jnp.exp(l - LMAX)
        s_ref[...] += jnp.sum(e, axis=1, keepdims=True)
        ebf_all[pl.ds(i, 1)] = e.astype(jnp.bfloat16)[None]
        est = pltpu.make_async_copy(
            ebf_all.at[i], ebf_hbm.at[:, pl.ds(i * CH, CH)],
            estore_sems.at[i])
        est.start()
        rdma = pltpu.make_async_remote_copy(
            src_ref=ebf_all.at[i],
            dst_ref=recv_hbm.at[:, pl.ds(i * CH, CH)],
            send_sem=send_sems.at[i], recv_sem=recv_sems.at[i],
            device_id=partner, device_id_type=pl.DeviceIdType.MESH)
        rdma.start()

        @pl.when(i == NC - 1)
        def _():
            st_rdma = pltpu.make_async_remote_copy(
                src_ref=s_ref, dst_ref=stats_recv,
                send_sem=st_send_sem, recv_sem=st_recv_sem,
                device_id=partner, device_id_type=pl.DeviceIdType.MESH)
            st_rdma.start()
            st_rdma.wait()
            s_out_vmem[...] = s_ref[...] + stats_recv[...]
            sst = pltpu.make_async_copy(s_out_vmem, stot_ref, sstore_sem)
            sst.start()
            sst.wait()

            def chunk_desc(c):
                return pltpu.make_async_remote_copy(
                    src_ref=ebf_all.at[c],
                    dst_ref=recv_hbm.at[:, pl.ds(c * CH, CH)],
                    send_sem=send_sems.at[c], recv_sem=recv_sems.at[c],
                    device_id=partner, device_id_type=pl.DeviceIdType.MESH)

            for c in range(NC):
                chunk_desc(c).wait_recv()
                chunk_desc(c).wait_send()
                pltpu.make_async_copy(
                    ebf_all.at[c], ebf_hbm.at[:, pl.ds(c * CH, CH)],
                    estore_sems.at[c]).wait()

    rcv, ebf, s_tot = pl.pallas_call(
        body,
        grid=(NC,),
        out_shape=(
            jax.ShapeDtypeStruct((T, HALF), jnp.bfloat16),
            jax.ShapeDtypeStruct((T, HALF), jnp.bfloat16),
            jax.ShapeDtypeStruct((T, 1), jnp.float32),
        ),
        in_specs=[
            pl.BlockSpec((T, D), lambda i: (0, 0)),
            pl.BlockSpec((D, CH), lambda i: (0, i)),
        ],
        out_specs=(
            pl.BlockSpec(memory_space=pl.ANY),
            pl.BlockSpec(memory_space=pl.ANY),
            pl.BlockSpec(memory_space=pl.ANY),
        ),
        scratch_shapes=[
            pltpu.VMEM((T, 1), jnp.float32),
            pltpu.VMEM((NC, T, CH), jnp.bfloat16),
            pltpu.VMEM((T, 1), jnp.float32),
            pltpu.VMEM((T, 1), jnp.float32),
            pltpu.SemaphoreType.DMA((NC,)),
            pltpu.SemaphoreType.DMA((NC,)),
            pltpu.SemaphoreType.DMA,
            pltpu.SemaphoreType.DMA,
            pltpu.SemaphoreType.DMA((NC,)),
            pltpu.SemaphoreType.DMA,
        ],
        compiler_params=pltpu.CompilerParams(
            collective_id=0,
            dimension_semantics=("arbitrary",),
            vmem_limit_bytes=60 * 1024 * 1024),
    )(xb, W)

    inv = 1.0 / s_tot
    mine = ebf.astype(jnp.float32) * inv
    theirs = rcv.astype(jnp.float32) * inv
    zi = lax.axis_index("z")
    on_left = (zi == 0)
    left = jnp.where(on_left, mine, theirs)
    right = jnp.where(on_left, theirs, mine)
    return jnp.concatenate([left, right], axis=1)


# device time: 448876 ns/iter; 1.1963x vs baseline; 1.1963x over previous
import jax
import jax.numpy as jnp
from jax import lax
from jax.experimental import pallas as pl
from jax.experimental.pallas import tpu as pltpu

T = 1024
D = 2048
HALF = 16384
CH = 512
NC = HALF // CH
LMAX = 8.0


def kernel(x, W):
    xb = x.astype(jnp.bfloat16)

    def body(xb_ref, wblk, combined, stot_ref,
             stats_recv, ebf_all, s_ref, s_out_vmem,
             send_sems, recv_sems, st_send_sem, st_recv_sem,
             estore_sems, sstore_sem):
        i = pl.program_id(0)
        xi = lax.axis_index("x")
        yi = lax.axis_index("y")
        zi = lax.axis_index("z")
        partner = (xi, yi, 1 - zi)
        mycol = zi * HALF

        @pl.when(i == 0)
        def _():
            barrier = pltpu.get_barrier_semaphore()
            pl.semaphore_signal(barrier, inc=1, device_id=partner,
                                device_id_type=pl.DeviceIdType.MESH)
            pl.semaphore_wait(barrier, 1)
            s_ref[...] = jnp.zeros((T, 1), jnp.float32)

        wb = wblk[...].astype(jnp.bfloat16)
        l = jnp.dot(xb_ref[...], wb, preferred_element_type=jnp.float32)
        e = jnp.exp(l - LMAX)
        s_ref[...] += jnp.sum(e, axis=1, keepdims=True)
        ebf_all[pl.ds(i, 1)] = e.astype(jnp.bfloat16)[None]
        est = pltpu.make_async_copy(
            ebf_all.at[i], combined.at[:, pl.ds(mycol + i * CH, CH)],
            estore_sems.at[i])
        est.start()
        rdma = pltpu.make_async_remote_copy(
            src_ref=ebf_all.at[i],
            dst_ref=combined.at[:, pl.ds(mycol + i * CH, CH)],
            send_sem=send_sems.at[i], recv_sem=recv_sems.at[i],
            device_id=partner, device_id_type=pl.DeviceIdType.MESH)
        rdma.start()

        @pl.when(i == NC - 1)
        def _():
            st_rdma = pltpu.make_async_remote_copy(
                src_ref=s_ref, dst_ref=stats_recv,
                send_sem=st_send_sem, recv_sem=st_recv_sem,
                device_id=partner, device_id_type=pl.DeviceIdType.MESH)
            st_rdma.start()
            st_rdma.wait()
            s_out_vmem[...] = s_ref[...] + stats_recv[...]
            sst = pltpu.make_async_copy(s_out_vmem, stot_ref, sstore_sem)
            sst.start()
            sst.wait()

            def chunk_desc(c):
                return pltpu.make_async_remote_copy(
                    src_ref=ebf_all.at[c],
                    dst_ref=combined.at[:, pl.ds(mycol + c * CH, CH)],
                    send_sem=send_sems.at[c], recv_sem=recv_sems.at[c],
                    device_id=partner, device_id_type=pl.DeviceIdType.MESH)

            for c in range(NC):
                chunk_desc(c).wait_recv()
                chunk_desc(c).wait_send()
                pltpu.make_async_copy(
                    ebf_all.at[c],
                    combined.at[:, pl.ds(mycol + c * CH, CH)],
                    estore_sems.at[c]).wait()

    combined, s_tot = pl.pallas_call(
        body,
        grid=(NC,),
        out_shape=(
            jax.ShapeDtypeStruct((T, 2 * HALF), jnp.bfloat16),
            jax.ShapeDtypeStruct((T, 1), jnp.float32),
        ),
        in_specs=[
            pl.BlockSpec((T, D), lambda i: (0, 0)),
            pl.BlockSpec((D, CH), lambda i: (0, i)),
        ],
        out_specs=(
            pl.BlockSpec(memory_space=pl.ANY),
            pl.BlockSpec(memory_space=pl.ANY),
        ),
        scratch_shapes=[
            pltpu.VMEM((T, 1), jnp.float32),
            pltpu.VMEM((NC, T, CH), jnp.bfloat16),
            pltpu.VMEM((T, 1), jnp.float32),
            pltpu.VMEM((T, 1), jnp.float32),
            pltpu.SemaphoreType.DMA((NC,)),
            pltpu.SemaphoreType.DMA((NC,)),
            pltpu.SemaphoreType.DMA,
            pltpu.SemaphoreType.DMA,
            pltpu.SemaphoreType.DMA((NC,)),
            pltpu.SemaphoreType.DMA,
        ],
        compiler_params=pltpu.CompilerParams(
            collective_id=0,
            dimension_semantics=("arbitrary",),
            vmem_limit_bytes=60 * 1024 * 1024),
    )(xb, W)

    return combined.astype(jnp.float32) * (1.0 / s_tot)


# device time: 445574 ns/iter; 1.2052x vs baseline; 1.0074x over previous
import jax
import jax.numpy as jnp
from jax import lax
from jax.experimental import pallas as pl
from jax.experimental.pallas import tpu as pltpu

T = 1024
D = 2048
HALF = 16384
CH = 512
NC = HALF // CH
LMAX = 8.0


def kernel(x, W):
    def body(x_ref, wblk, combined, stot_ref,
             stats_recv, ebf_all, xb_vmem, s_ref, s_out_vmem,
             send_sems, recv_sems, st_send_sem, st_recv_sem,
             estore_sems, sstore_sem):
        i = pl.program_id(0)
        xi = lax.axis_index("x")
        yi = lax.axis_index("y")
        zi = lax.axis_index("z")
        partner = (xi, yi, 1 - zi)
        mycol = zi * HALF

        @pl.when(i == 0)
        def _():
            barrier = pltpu.get_barrier_semaphore()
            pl.semaphore_signal(barrier, inc=1, device_id=partner,
                                device_id_type=pl.DeviceIdType.MESH)
            pl.semaphore_wait(barrier, 1)
            s_ref[...] = jnp.zeros((T, 1), jnp.float32)
            xb_vmem[...] = x_ref[...].astype(jnp.bfloat16)

        wb = wblk[...].astype(jnp.bfloat16)
        l = jnp.dot(xb_vmem[...], wb, preferred_element_type=jnp.float32)
        e = jnp.exp(l - LMAX)
        s_ref[...] += jnp.sum(e, axis=1, keepdims=True)
        ebf_all[pl.ds(i, 1)] = e.astype(jnp.bfloat16)[None]
        est = pltpu.make_async_copy(
            ebf_all.at[i], combined.at[:, pl.ds(mycol + i * CH, CH)],
            estore_sems.at[i])
        est.start()
        rdma = pltpu.make_async_remote_copy(
            src_ref=ebf_all.at[i],
            dst_ref=combined.at[:, pl.ds(mycol + i * CH, CH)],
            send_sem=send_sems.at[i], recv_sem=recv_sems.at[i],
            device_id=partner, device_id_type=pl.DeviceIdType.MESH)
        rdma.start()

        @pl.when(i == NC - 1)
        def _():
            st_rdma = pltpu.make_async_remote_copy(
                src_ref=s_ref, dst_ref=stats_recv,
                send_sem=st_send_sem, recv_sem=st_recv_sem,
                device_id=partner, device_id_type=pl.DeviceIdType.MESH)
            st_rdma.start()
            st_rdma.wait()
            s_out_vmem[...] = s_ref[...] + stats_recv[...]
            sst = pltpu.make_async_copy(s_out_vmem, stot_ref, sstore_sem)
            sst.start()
            sst.wait()

            def chunk_desc(c):
                return pltpu.make_async_remote_copy(
                    src_ref=ebf_all.at[c],
                    dst_ref=combined.at[:, pl.ds(mycol + c * CH, CH)],
                    send_sem=send_sems.at[c], recv_sem=recv_sems.at[c],
                    device_id=partner, device_id_type=pl.DeviceIdType.MESH)

            for c in range(NC):
                chunk_desc(c).wait_recv()
                chunk_desc(c).wait_send()
                pltpu.make_async_copy(
                    ebf_all.at[c],
                    combined.at[:, pl.ds(mycol + c * CH, CH)],
                    estore_sems.at[c]).wait()

    combined, s_tot = pl.pallas_call(
        body,
        grid=(NC,),
        out_shape=(
            jax.ShapeDtypeStruct((T, 2 * HALF), jnp.bfloat16),
            jax.ShapeDtypeStruct((T, 1), jnp.float32),
        ),
        in_specs=[
            pl.BlockSpec((T, D), lambda i: (0, 0)),
            pl.BlockSpec((D, CH), lambda i: (0, i)),
        ],
        out_specs=(
            pl.BlockSpec(memory_space=pl.ANY),
            pl.BlockSpec(memory_space=pl.ANY),
        ),
        scratch_shapes=[
            pltpu.VMEM((T, 1), jnp.float32),
            pltpu.VMEM((NC, T, CH), jnp.bfloat16),
            pltpu.VMEM((T, D), jnp.bfloat16),
            pltpu.VMEM((T, 1), jnp.float32),
            pltpu.VMEM((T, 1), jnp.float32),
            pltpu.SemaphoreType.DMA((NC,)),
            pltpu.SemaphoreType.DMA((NC,)),
            pltpu.SemaphoreType.DMA,
            pltpu.SemaphoreType.DMA,
            pltpu.SemaphoreType.DMA((NC,)),
            pltpu.SemaphoreType.DMA,
        ],
        compiler_params=pltpu.CompilerParams(
            collective_id=0,
            dimension_semantics=("arbitrary",),
            vmem_limit_bytes=60 * 1024 * 1024),
    )(x, W)

    return combined.astype(jnp.float32) * (1.0 / s_tot)
